# baseline (speedup 1.0000x reference)
import jax
import jax.numpy as jnp
from jax.experimental import pallas as pl
from jax.experimental.pallas import tpu as pltpu

_N = 10000
_G = 100
_NPG = 100
_E_PER = 3200
_D = 128
_K = 800
_OUT = 10
_B = 20
_HC = 128
_SROWS = 32
_SCOLS = 128
_SORT_N = _SROWS * _SCOLS
_NEG = -3.0e38


def _partner(v, j):
    if j % _SROWS == 0:
        m = j // _SROWS
        return jnp.roll(v, -m, axis=2), jnp.roll(v, m, axis=2)
    return jnp.roll(v, -j, axis=0), jnp.roll(v, j, axis=0)


def _bitonic_desc(v):
    r = jax.lax.broadcasted_iota(jnp.int32, (_SROWS, _B, _SCOLS), 0)
    c = jax.lax.broadcasted_iota(jnp.int32, (_SROWS, _B, _SCOLS), 2)
    i = c * _SROWS + r
    bitj_m = {a: (i & (1 << a)) != 0 for a in range(12)}
    up_m = {a: (i & (1 << a)) == 0 for a in range(1, 13)}
    k = 2
    while k <= _SORT_N:
        j = k // 2
        while j >= 1:
            bitj = bitj_m[j.bit_length() - 1]
            fwd, bwd = _partner(v, j)
            pv = jnp.where(bitj, bwd, fwd)
            take_max = jnp.equal(up_m[k.bit_length() - 1],
                                 jnp.logical_not(bitj))
            mn = jnp.minimum(v, pv)
            mx = jnp.maximum(v, pv)
            v = jnp.where(take_max, mx, mn)
            j //= 2
        k *= 2
    return v


def _body(ht_ref, rl_ref, cl_ref, w1t_ref, b1_ref, w2_ref, b2_ref,
          wct_ref, bct_ref, sorted_ref, pred_ref):
    nodes = jax.lax.broadcasted_iota(jnp.int32, (_NPG, _E_PER), 0)
    sTs = []
    OrTs = []
    for b in range(_B):
        hgT_bf = ht_ref[b].astype(jnp.bfloat16)
        r2 = rl_ref[b]
        c2 = cl_ref[b]
        OrT = (nodes == r2).astype(jnp.bfloat16)
        OcT = (nodes == c2).astype(jnp.bfloat16)
        hrT = jnp.dot(hgT_bf, OrT, preferred_element_type=jnp.float32)
        hcT = jnp.dot(hgT_bf, OcT, preferred_element_type=jnp.float32)
        catT = jnp.concatenate([hrT, hcT], axis=0).astype(jnp.bfloat16)
        sT = b2_ref[0, 0]
        for h0 in range(0, 4 * _D, _HC):
            zc = jnp.dot(w1t_ref[pl.ds(h0, _HC), :], catT,
                         preferred_element_type=jnp.float32)
            zc = zc + b1_ref[pl.ds(h0, _HC), :]
            zrc = jnp.maximum(zc, 0.0).astype(jnp.bfloat16)
            sT = sT + jnp.dot(w2_ref[:, pl.ds(h0, _HC)], zrc,
                              preferred_element_type=jnp.float32)
        sTs.append(sT)
        OrTs.append(OrT)
    v0 = jnp.concatenate(
        [jnp.concatenate(
            [sT.reshape(_E_PER // _SCOLS, _SCOLS),
             jnp.full((_SROWS - _E_PER // _SCOLS, _SCOLS), _NEG, jnp.float32)],
            axis=0)[:, None, :] for sT in sTs],
        axis=1)
    vs = _bitonic_desc(v0)
    sorted_ref[0] = vs.reshape(_SROWS, _B * _SCOLS)
    for b in range(_B):
        sT = sTs[b]
        OrT = OrTs[b]
        t = vs[(_K - 1) % _SROWS, b, (_K - 1) // _SROWS]
        gt = sT > t
        cnt = jnp.sum(gt.astype(jnp.float32))
        ties = (sT == t).astype(jnp.float32)
        nt = jnp.sum(ties)
        wsel = jnp.where(gt, sT, 0.0)
        u_main = jnp.sum(OrT * wsel, axis=1, keepdims=True)
        u_tie = jnp.sum(OrT * ties, axis=1, keepdims=True)
        u = u_main + (t * (_K - cnt) / nt) * u_tie
        pooled = jnp.dot(ht_ref[b], u, preferred_element_type=jnp.float32,
                         precision=jax.lax.Precision.HIGHEST) / _K
        pred = jnp.dot(wct_ref[...], pooled.astype(jnp.bfloat16),
                       preferred_element_type=jnp.float32)
        pred_ref[b] = pred + bct_ref[...]


def kernel(h, edge_index, W1, b1, W2, b2, Wc, bc):
    row = edge_index[0].astype(jnp.int32)
    col = edge_index[1].astype(jnp.int32)
    rl = (row % _NPG).reshape(_G, 1, _E_PER)
    cl = (col % _NPG).reshape(_G, 1, _E_PER)
    hT = h.reshape(_G, _NPG, _D).transpose(0, 2, 1)
    W1T = W1.T.astype(jnp.bfloat16)
    b1c = b1.reshape(4 * _D, 1)
    w2r = W2.reshape(1, 4 * _D).astype(jnp.bfloat16)
    b2c = b2.reshape(1, 1)
    WcT = Wc.T.astype(jnp.bfloat16)
    bcc = bc.reshape(_OUT, 1)

    grid = (_G // _B,)
    sorted_out, pred_out = pl.pallas_call(
        _body,
        grid=grid,
        in_specs=[
            pl.BlockSpec((_B, _D, _NPG), lambda g: (g, 0, 0)),
            pl.BlockSpec((_B, 1, _E_PER), lambda g: (g, 0, 0)),
            pl.BlockSpec((_B, 1, _E_PER), lambda g: (g, 0, 0)),
            pl.BlockSpec((4 * _D, 2 * _D), lambda g: (0, 0)),
            pl.BlockSpec((4 * _D, 1), lambda g: (0, 0)),
            pl.BlockSpec((1, 4 * _D), lambda g: (0, 0)),
            pl.BlockSpec((1, 1), lambda g: (0, 0)),
            pl.BlockSpec((_OUT, _D), lambda g: (0, 0)),
            pl.BlockSpec((_OUT, 1), lambda g: (0, 0)),
        ],
        out_specs=[
            pl.BlockSpec((1, _SROWS, _B * _SCOLS), lambda g: (g, 0, 0)),
            pl.BlockSpec((_B, _OUT, 1), lambda g: (g, 0, 0)),
        ],
        out_shape=[
            jax.ShapeDtypeStruct((_G // _B, _SROWS, _B * _SCOLS), jnp.float32),
            jax.ShapeDtypeStruct((_G, _OUT, 1), jnp.float32),
        ],
        compiler_params=pltpu.CompilerParams(
            dimension_semantics=("parallel",),
        ),
    )(hT, rl, cl, W1T, b1c, w2r, b2c, WcT, bcc)

    srt = (sorted_out.reshape(_G // _B, _SROWS, _B, _SCOLS)
           .transpose(0, 2, 3, 1).reshape(_G, _SORT_N))
    causal_edge_weight = srt[:, :_K]
    spu_edge_weight = -srt[:, _K:_E_PER]
    causal_pred = pred_out.reshape(_G, _OUT)
    return (causal_pred, causal_edge_weight, spu_edge_weight)


# speedup vs baseline: 1.2096x; 1.2096x over previous
import jax
import jax.numpy as jnp
from jax.experimental import pallas as pl
from jax.experimental.pallas import tpu as pltpu

_N = 10000
_G = 100
_NPG = 100
_E_PER = 3200
_D = 128
_K = 800
_OUT = 10
_B = 10
_HC = 256
_SROWS = 32
_SCOLS = 128
_SORT_N = _SROWS * _SCOLS
_NEG = -3.0e38


def _partner(v, j):
    if j % _SROWS == 0:
        m = j // _SROWS
        return jnp.roll(v, -m, axis=2), jnp.roll(v, m, axis=2)
    return jnp.roll(v, -j, axis=0), jnp.roll(v, j, axis=0)


def _bitonic_desc(v):
    r = jax.lax.broadcasted_iota(jnp.int32, (_SROWS, _B, _SCOLS), 0)
    c = jax.lax.broadcasted_iota(jnp.int32, (_SROWS, _B, _SCOLS), 2)
    i = c * _SROWS + r
    bitj_m = {a: (i & (1 << a)) != 0 for a in range(12)}
    up_m = {a: (i & (1 << a)) == 0 for a in range(1, 13)}
    k = 2
    while k <= _SORT_N:
        j = k // 2
        while j >= 1:
            bitj = bitj_m[j.bit_length() - 1]
            fwd, bwd = _partner(v, j)
            pv = jnp.where(bitj, bwd, fwd)
            take_max = jnp.equal(up_m[k.bit_length() - 1],
                                 jnp.logical_not(bitj))
            mn = jnp.minimum(v, pv)
            mx = jnp.maximum(v, pv)
            v = jnp.where(take_max, mx, mn)
            j //= 2
        k *= 2
    return v


def _body(ht_ref, rl_ref, cl_ref, w1t_ref, b1_ref, w2_ref, b2_ref,
          wct_ref, bct_ref, sorted_ref, pred_ref):
    nodes = jax.lax.broadcasted_iota(jnp.int32, (_NPG, _E_PER), 0)
    sTs = []
    OrTs = []
    for b in range(_B):
        hgT_bf = ht_ref[b].astype(jnp.bfloat16)
        r2 = rl_ref[b]
        c2 = cl_ref[b]
        OrT = (nodes == r2).astype(jnp.bfloat16)
        OcT = (nodes == c2).astype(jnp.bfloat16)
        hrT = jnp.dot(hgT_bf, OrT, preferred_element_type=jnp.float32)
        hcT = jnp.dot(hgT_bf, OcT, preferred_element_type=jnp.float32)
        catT = jnp.concatenate([hrT, hcT], axis=0).astype(jnp.bfloat16)
        sT = b2_ref[0, 0]
        for h0 in range(0, 4 * _D, _HC):
            zc = jnp.dot(w1t_ref[pl.ds(h0, _HC), :], catT,
                         preferred_element_type=jnp.float32)
            zc = zc + b1_ref[pl.ds(h0, _HC), :]
            zrc = jnp.maximum(zc, 0.0).astype(jnp.bfloat16)
            sT = sT + jnp.dot(w2_ref[:, pl.ds(h0, _HC)], zrc,
                              preferred_element_type=jnp.float32)
        sTs.append(sT)
        OrTs.append(OrT)
    v0 = jnp.concatenate(
        [jnp.concatenate(
            [sT.reshape(_E_PER // _SCOLS, _SCOLS),
             jnp.full((_SROWS - _E_PER // _SCOLS, _SCOLS), _NEG, jnp.float32)],
            axis=0)[:, None, :] for sT in sTs],
        axis=1)
    vs = _bitonic_desc(v0)
    sorted_ref[0] = vs.reshape(_SROWS, _B * _SCOLS)
    for b in range(_B):
        sT = sTs[b]
        OrT = OrTs[b]
        t = vs[(_K - 1) % _SROWS, b, (_K - 1) // _SROWS]
        gt = sT > t
        cnt = jnp.sum(gt.astype(jnp.float32))
        ties = (sT == t).astype(jnp.float32)
        nt = jnp.sum(ties)
        wsel = jnp.where(gt, sT, 0.0)
        u_main = jnp.sum(OrT * wsel, axis=1, keepdims=True)
        u_tie = jnp.sum(OrT * ties, axis=1, keepdims=True)
        u = u_main + (t * (_K - cnt) / nt) * u_tie
        pooled = jnp.dot(ht_ref[b], u, preferred_element_type=jnp.float32,
                         precision=jax.lax.Precision.HIGHEST) / _K
        pred = jnp.dot(wct_ref[...], pooled.astype(jnp.bfloat16),
                       preferred_element_type=jnp.float32)
        pred_ref[b] = pred + bct_ref[...]


def kernel(h, edge_index, W1, b1, W2, b2, Wc, bc):
    row = edge_index[0].astype(jnp.int32)
    col = edge_index[1].astype(jnp.int32)
    rl = (row % _NPG).reshape(_G, 1, _E_PER)
    cl = (col % _NPG).reshape(_G, 1, _E_PER)
    hT = h.reshape(_G, _NPG, _D).transpose(0, 2, 1)
    W1T = W1.T.astype(jnp.bfloat16)
    b1c = b1.reshape(4 * _D, 1)
    w2r = W2.reshape(1, 4 * _D).astype(jnp.bfloat16)
    b2c = b2.reshape(1, 1)
    WcT = Wc.T.astype(jnp.bfloat16)
    bcc = bc.reshape(_OUT, 1)

    grid = (_G // _B,)
    sorted_out, pred_out = pl.pallas_call(
        _body,
        grid=grid,
        in_specs=[
            pl.BlockSpec((_B, _D, _NPG), lambda g: (g, 0, 0)),
            pl.BlockSpec((_B, 1, _E_PER), lambda g: (g, 0, 0)),
            pl.BlockSpec((_B, 1, _E_PER), lambda g: (g, 0, 0)),
            pl.BlockSpec((4 * _D, 2 * _D), lambda g: (0, 0)),
            pl.BlockSpec((4 * _D, 1), lambda g: (0, 0)),
            pl.BlockSpec((1, 4 * _D), lambda g: (0, 0)),
            pl.BlockSpec((1, 1), lambda g: (0, 0)),
            pl.BlockSpec((_OUT, _D), lambda g: (0, 0)),
            pl.BlockSpec((_OUT, 1), lambda g: (0, 0)),
        ],
        out_specs=[
            pl.BlockSpec((1, _SROWS, _B * _SCOLS), lambda g: (g, 0, 0)),
            pl.BlockSpec((_B, _OUT, 1), lambda g: (g, 0, 0)),
        ],
        out_shape=[
            jax.ShapeDtypeStruct((_G // _B, _SROWS, _B * _SCOLS), jnp.float32),
            jax.ShapeDtypeStruct((_G, _OUT, 1), jnp.float32),
        ],
        compiler_params=pltpu.CompilerParams(
            dimension_semantics=("parallel",),
        ),
    )(hT, rl, cl, W1T, b1c, w2r, b2c, WcT, bcc)

    srt = (sorted_out.reshape(_G // _B, _SROWS, _B, _SCOLS)
           .transpose(0, 2, 3, 1).reshape(_G, _SORT_N))
    causal_edge_weight = srt[:, :_K]
    spu_edge_weight = -srt[:, _K:_E_PER]
    causal_pred = pred_out.reshape(_G, _OUT)
    return (causal_pred, causal_edge_weight, spu_edge_weight)


# speedup vs baseline: 1.2397x; 1.0249x over previous
import jax
import jax.numpy as jnp
from jax.experimental import pallas as pl
from jax.experimental.pallas import tpu as pltpu

_N = 10000
_G = 100
_NPG = 100
_E_PER = 3200
_D = 128
_K = 800
_OUT = 10
_B = 10
_HC = 512
_SROWS = 32
_SCOLS = 128
_SORT_N = _SROWS * _SCOLS
_NEG = -3.0e38


def _partner(v, j):
    if j % _SROWS == 0:
        m = j // _SROWS
        return jnp.roll(v, -m, axis=2), jnp.roll(v, m, axis=2)
    return jnp.roll(v, -j, axis=0), jnp.roll(v, j, axis=0)


def _bitonic_desc(v):
    r = jax.lax.broadcasted_iota(jnp.int32, (_SROWS, _B, _SCOLS), 0)
    c = jax.lax.broadcasted_iota(jnp.int32, (_SROWS, _B, _SCOLS), 2)
    i = c * _SROWS + r
    bitj_m = {a: (i & (1 << a)) != 0 for a in range(12)}
    up_m = {a: (i & (1 << a)) == 0 for a in range(1, 13)}
    k = 2
    while k <= _SORT_N:
        j = k // 2
        while j >= 1:
            bitj = bitj_m[j.bit_length() - 1]
            fwd, bwd = _partner(v, j)
            pv = jnp.where(bitj, bwd, fwd)
            take_max = jnp.equal(up_m[k.bit_length() - 1],
                                 jnp.logical_not(bitj))
            mn = jnp.minimum(v, pv)
            mx = jnp.maximum(v, pv)
            v = jnp.where(take_max, mx, mn)
            j //= 2
        k *= 2
    return v


def _body(ht_ref, rl_ref, cl_ref, w1t_ref, b1_ref, w2_ref, b2_ref,
          wct_ref, bct_ref, sorted_ref, pred_ref):
    nodes = jax.lax.broadcasted_iota(jnp.int32, (_NPG, _E_PER), 0)
    sTs = []
    OrTs = []
    for b in range(_B):
        hgT_bf = ht_ref[b].astype(jnp.bfloat16)
        r2 = rl_ref[b]
        c2 = cl_ref[b]
        OrT = (nodes == r2).astype(jnp.bfloat16)
        OcT = (nodes == c2).astype(jnp.bfloat16)
        hrT = jnp.dot(hgT_bf, OrT, preferred_element_type=jnp.float32)
        hcT = jnp.dot(hgT_bf, OcT, preferred_element_type=jnp.float32)
        catT = jnp.concatenate([hrT, hcT], axis=0).astype(jnp.bfloat16)
        sT = b2_ref[0, 0]
        for h0 in range(0, 4 * _D, _HC):
            zc = jnp.dot(w1t_ref[pl.ds(h0, _HC), :], catT,
                         preferred_element_type=jnp.float32)
            zc = zc + b1_ref[pl.ds(h0, _HC), :]
            zrc = jnp.maximum(zc, 0.0).astype(jnp.bfloat16)
            sT = sT + jnp.dot(w2_ref[:, pl.ds(h0, _HC)], zrc,
                              preferred_element_type=jnp.float32)
        sTs.append(sT)
        OrTs.append(OrT)
    v0 = jnp.concatenate(
        [jnp.concatenate(
            [sT.reshape(_E_PER // _SCOLS, _SCOLS),
             jnp.full((_SROWS - _E_PER // _SCOLS, _SCOLS), _NEG, jnp.float32)],
            axis=0)[:, None, :] for sT in sTs],
        axis=1)
    vs = _bitonic_desc(v0)
    sorted_ref[0] = vs.reshape(_SROWS, _B * _SCOLS)
    for b in range(_B):
        sT = sTs[b]
        OrT = OrTs[b]
        t = vs[(_K - 1) % _SROWS, b, (_K - 1) // _SROWS]
        gt = sT > t
        cnt = jnp.sum(gt.astype(jnp.float32))
        ties = (sT == t).astype(jnp.float32)
        nt = jnp.sum(ties)
        wsel = jnp.where(gt, sT, 0.0)
        u_main = jnp.sum(OrT * wsel, axis=1, keepdims=True)
        u_tie = jnp.sum(OrT * ties, axis=1, keepdims=True)
        u = u_main + (t * (_K - cnt) / nt) * u_tie
        pooled = jnp.dot(ht_ref[b], u, preferred_element_type=jnp.float32,
                         precision=jax.lax.Precision.HIGHEST) / _K
        pred = jnp.dot(wct_ref[...], pooled.astype(jnp.bfloat16),
                       preferred_element_type=jnp.float32)
        pred_ref[b] = pred + bct_ref[...]


def kernel(h, edge_index, W1, b1, W2, b2, Wc, bc):
    row = edge_index[0].astype(jnp.int32)
    col = edge_index[1].astype(jnp.int32)
    rl = (row % _NPG).reshape(_G, 1, _E_PER)
    cl = (col % _NPG).reshape(_G, 1, _E_PER)
    hT = h.reshape(_G, _NPG, _D).transpose(0, 2, 1)
    W1T = W1.T.astype(jnp.bfloat16)
    b1c = b1.reshape(4 * _D, 1)
    w2r = W2.reshape(1, 4 * _D).astype(jnp.bfloat16)
    b2c = b2.reshape(1, 1)
    WcT = Wc.T.astype(jnp.bfloat16)
    bcc = bc.reshape(_OUT, 1)

    grid = (_G // _B,)
    sorted_out, pred_out = pl.pallas_call(
        _body,
        grid=grid,
        in_specs=[
            pl.BlockSpec((_B, _D, _NPG), lambda g: (g, 0, 0)),
            pl.BlockSpec((_B, 1, _E_PER), lambda g: (g, 0, 0)),
            pl.BlockSpec((_B, 1, _E_PER), lambda g: (g, 0, 0)),
            pl.BlockSpec((4 * _D, 2 * _D), lambda g: (0, 0)),
            pl.BlockSpec((4 * _D, 1), lambda g: (0, 0)),
            pl.BlockSpec((1, 4 * _D), lambda g: (0, 0)),
            pl.BlockSpec((1, 1), lambda g: (0, 0)),
            pl.BlockSpec((_OUT, _D), lambda g: (0, 0)),
            pl.BlockSpec((_OUT, 1), lambda g: (0, 0)),
        ],
        out_specs=[
            pl.BlockSpec((1, _SROWS, _B * _SCOLS), lambda g: (g, 0, 0)),
            pl.BlockSpec((_B, _OUT, 1), lambda g: (g, 0, 0)),
        ],
        out_shape=[
            jax.ShapeDtypeStruct((_G // _B, _SROWS, _B * _SCOLS), jnp.float32),
            jax.ShapeDtypeStruct((_G, _OUT, 1), jnp.float32),
        ],
        compiler_params=pltpu.CompilerParams(
            dimension_semantics=("parallel",),
        ),
    )(hT, rl, cl, W1T, b1c, w2r, b2c, WcT, bcc)

    srt = (sorted_out.reshape(_G // _B, _SROWS, _B, _SCOLS)
           .transpose(0, 2, 3, 1).reshape(_G, _SORT_N))
    causal_edge_weight = srt[:, :_K]
    spu_edge_weight = -srt[:, _K:_E_PER]
    causal_pred = pred_out.reshape(_G, _OUT)
    return (causal_pred, causal_edge_weight, spu_edge_weight)


# speedup vs baseline: 1.2416x; 1.0016x over previous
import jax
import jax.numpy as jnp
from jax.experimental import pallas as pl
from jax.experimental.pallas import tpu as pltpu

_N = 10000
_G = 100
_NPG = 100
_E_PER = 3200
_D = 128
_K = 800
_OUT = 10
_B = 10
_HC = 512
_SROWS = 32
_SCOLS = 128
_SORT_N = _SROWS * _SCOLS
_NEG = -3.0e38


def _partner(v, j):
    if j % _SROWS == 0:
        m = j // _SROWS
        return jnp.roll(v, -m, axis=2), jnp.roll(v, m, axis=2)
    return jnp.roll(v, -j, axis=0), jnp.roll(v, j, axis=0)


def _bitonic_desc(v):
    r = jax.lax.broadcasted_iota(jnp.int32, (_SROWS, _B, _SCOLS), 0)
    c = jax.lax.broadcasted_iota(jnp.int32, (_SROWS, _B, _SCOLS), 2)
    i = c * _SROWS + r
    bitj_m = {a: (i & (1 << a)) != 0 for a in range(12)}
    up_m = {a: (i & (1 << a)) == 0 for a in range(1, 13)}
    k = 2
    while k <= _SORT_N:
        j = k // 2
        while j >= 1:
            bitj = bitj_m[j.bit_length() - 1]
            fwd, bwd = _partner(v, j)
            pv = jnp.where(bitj, bwd, fwd)
            take_max = jnp.equal(up_m[k.bit_length() - 1],
                                 jnp.logical_not(bitj))
            mn = jnp.minimum(v, pv)
            mx = jnp.maximum(v, pv)
            v = jnp.where(take_max, mx, mn)
            j //= 2
        k *= 2
    return v


def _body(ht_ref, rl_ref, cl_ref, w1t_ref, b1_ref, w2_ref, b2_ref,
          wct_ref, bct_ref, sorted_ref, pred_ref):
    nodes = jax.lax.broadcasted_iota(jnp.int32, (_NPG, _E_PER), 0)
    sTs = []
    OrTs = []
    for b in range(_B):
        hgT_bf = ht_ref[b].astype(jnp.bfloat16)
        r2 = rl_ref[b]
        c2 = cl_ref[b]
        OrT = (nodes == r2).astype(jnp.bfloat16)
        OcT = (nodes == c2).astype(jnp.bfloat16)
        hrT = jnp.dot(hgT_bf, OrT, preferred_element_type=jnp.float32)
        hcT = jnp.dot(hgT_bf, OcT, preferred_element_type=jnp.float32)
        catT = jnp.concatenate([hrT, hcT], axis=0).astype(jnp.bfloat16)
        sT = b2_ref[0, 0]
        for h0 in range(0, 4 * _D, _HC):
            zc = jnp.dot(w1t_ref[pl.ds(h0, _HC), :], catT,
                         preferred_element_type=jnp.float32)
            zc = zc + b1_ref[pl.ds(h0, _HC), :]
            zrc = jnp.maximum(zc, 0.0).astype(jnp.bfloat16)
            sT = sT + jnp.dot(w2_ref[:, pl.ds(h0, _HC)], zrc,
                              preferred_element_type=jnp.float32)
        sTs.append(sT)
        OrTs.append(OrT)
    v0 = jnp.concatenate(
        [jnp.concatenate(
            [sT.reshape(_E_PER // _SCOLS, _SCOLS),
             jnp.full((_SROWS - _E_PER // _SCOLS, _SCOLS), _NEG, jnp.float32)],
            axis=0)[:, None, :] for sT in sTs],
        axis=1)
    vs = _bitonic_desc(v0)
    sorted_ref[0] = vs.reshape(_SROWS, _B * _SCOLS)
    for b in range(_B):
        sT = sTs[b]
        OrT = OrTs[b]
        t = vs[(_K - 1) % _SROWS, b, (_K - 1) // _SROWS]
        gt = sT > t
        cnt = jnp.sum(gt.astype(jnp.float32))
        ties = (sT == t).astype(jnp.float32)
        nt = jnp.sum(ties)
        wsel = jnp.where(gt, sT, 0.0)
        u_main = jnp.sum(OrT * wsel, axis=1, keepdims=True)
        u_tie = jnp.sum(OrT * ties, axis=1, keepdims=True)
        u = u_main + (t * (_K - cnt) / nt) * u_tie
        pooled = jnp.dot(ht_ref[b], u, preferred_element_type=jnp.float32,
                         precision=jax.lax.Precision.HIGHEST) / _K
        pred = jnp.dot(wct_ref[...], pooled.astype(jnp.bfloat16),
                       preferred_element_type=jnp.float32)
        pred_ref[b] = pred + bct_ref[...]


def kernel(h, edge_index, W1, b1, W2, b2, Wc, bc):
    row = edge_index[0].astype(jnp.int32)
    col = edge_index[1].astype(jnp.int32)
    rl = (row % _NPG).reshape(_G, 1, _E_PER)
    cl = (col % _NPG).reshape(_G, 1, _E_PER)
    hT = h.reshape(_G, _NPG, _D).transpose(0, 2, 1)
    W1T = W1.T.astype(jnp.bfloat16)
    b1c = b1.reshape(4 * _D, 1)
    w2r = W2.reshape(1, 4 * _D).astype(jnp.bfloat16)
    b2c = b2.reshape(1, 1)
    WcT = Wc.T.astype(jnp.bfloat16)
    bcc = bc.reshape(_OUT, 1)

    grid = (_G // _B,)
    sorted_out, pred_out = pl.pallas_call(
        _body,
        grid=grid,
        in_specs=[
            pl.BlockSpec((_B, _D, _NPG), lambda g: (g, 0, 0)),
            pl.BlockSpec((_B, 1, _E_PER), lambda g: (g, 0, 0)),
            pl.BlockSpec((_B, 1, _E_PER), lambda g: (g, 0, 0)),
            pl.BlockSpec((4 * _D, 2 * _D), lambda g: (0, 0)),
            pl.BlockSpec((4 * _D, 1), lambda g: (0, 0)),
            pl.BlockSpec((1, 4 * _D), lambda g: (0, 0)),
            pl.BlockSpec((1, 1), lambda g: (0, 0)),
            pl.BlockSpec((_OUT, _D), lambda g: (0, 0)),
            pl.BlockSpec((_OUT, 1), lambda g: (0, 0)),
        ],
        out_specs=[
            pl.BlockSpec((1, _SROWS, _B * _SCOLS), lambda g: (g, 0, 0)),
            pl.BlockSpec((_B, _OUT, 1), lambda g: (g, 0, 0)),
        ],
        out_shape=[
            jax.ShapeDtypeStruct((_G // _B, _SROWS, _B * _SCOLS), jnp.float32),
            jax.ShapeDtypeStruct((_G, _OUT, 1), jnp.float32),
        ],
        compiler_params=pltpu.CompilerParams(
            dimension_semantics=("arbitrary",),
        ),
    )(hT, rl, cl, W1T, b1c, w2r, b2c, WcT, bcc)

    srt = (sorted_out.reshape(_G // _B, _SROWS, _B, _SCOLS)
           .transpose(0, 2, 3, 1).reshape(_G, _SORT_N))
    causal_edge_weight = srt[:, :_K]
    spu_edge_weight = -srt[:, _K:_E_PER]
    causal_pred = pred_out.reshape(_G, _OUT)
    return (causal_pred, causal_edge_weight, spu_edge_weight)
